# Initial kernel scaffold; baseline (speedup 1.0000x reference)
import jax
import jax.numpy as jnp
from jax.experimental import pallas as pl


def kernel(x, pe):
    raise NotImplementedError("write your pallas kernel here")


# speedup vs baseline: 1.7227x; 1.7227x over previous
import functools

import jax
import jax.numpy as jnp
from jax import lax
from jax.experimental import pallas as pl
from jax.experimental.pallas import tpu as pltpu
from jax.experimental.pallas import tpu_sc as plsc

D = 256
NC = 2
NS = 16
NW = NC * NS
PER_W = 6272
WIN = 64


def _gather_kernel(pe_hbm, idx_hbm, out_hbm, idx_v, rows_v, sem):
    wid = lax.axis_index("s") * NC + lax.axis_index("c")
    rows_total = out_hbm.shape[0]
    base = wid * PER_W
    nrows = jnp.minimum(PER_W, rows_total - base)
    nwin = nrows // WIN

    def body(j, _):
        st = base + j * WIN
        pltpu.sync_copy(idx_hbm.at[pl.ds(st, WIN)], idx_v)
        pltpu.async_copy(pe_hbm.at[idx_v], rows_v, sem).wait()
        pltpu.sync_copy(rows_v, out_hbm.at[pl.ds(st, WIN), :])
        return 0

    lax.fori_loop(0, nwin, body, 0)


def _gather_rows(pe_ext, idx):
    rows = idx.shape[0]
    call = functools.partial(
        pl.kernel,
        out_type=jax.ShapeDtypeStruct((rows, D), jnp.float32),
        mesh=plsc.VectorSubcoreMesh(core_axis_name="c", subcore_axis_name="s"),
        scratch_types=[
            pltpu.VMEM((WIN,), jnp.int32),
            pltpu.VMEM((WIN, D), jnp.float32),
            pltpu.SemaphoreType.DMA,
        ],
    )(_gather_kernel)
    return call(pe_ext, idx)


def kernel(x, pe):
    rows = x.shape[0]
    n = rows // 2
    keys = x[:, 0] * 2 + x[:, 1]
    winner = (
        jnp.full((rows,), -1, jnp.int32)
        .at[keys]
        .set(lax.iota(jnp.int32, rows))
    )
    zrows = 64
    pe_ext = jnp.concatenate(
        [pe, jnp.zeros((zrows, pe.shape[1]), pe.dtype)], axis=0
    )
    k_iota = lax.iota(jnp.int32, rows)
    idx = jnp.where(winner >= 0, winner, rows + (k_iota & (zrows - 1)))
    out = _gather_rows(pe_ext, idx)
    return out.reshape(n, 2 * D)


# speedup vs baseline: 3.5451x; 2.0579x over previous
import functools

import jax
import jax.numpy as jnp
from jax import lax
from jax.experimental import pallas as pl
from jax.experimental.pallas import tpu as pltpu
from jax.experimental.pallas import tpu_sc as plsc

D = 256
NC = 2
NS = 16
NW = NC * NS
PER_W = 6272
WIN = 64
KWIN = 2000
MAXW = PER_W // WIN


def _sc_kernel(keys_hbm, pe_hbm, out_hbm,
               winner_v, kbuf, ibuf, inv_flat, zbuf, sem):
    wid = lax.axis_index("s") * NC + lax.axis_index("c")
    rows_total = out_hbm.shape[0]
    nkeys = keys_hbm.shape[0]
    base = wid * PER_W
    nrows = jnp.minimum(PER_W, rows_total - base)
    hi = base + nrows
    lanes = lax.iota(jnp.int32, 16)

    def init_body(t, _):
        winner_v[pl.ds(t * 16, 16)] = jnp.full((16,), -1, jnp.int32)
        return 0
    lax.fori_loop(0, PER_W // 16, init_body, 0)

    def key_window(w, _):
        pltpu.sync_copy(keys_hbm.at[pl.ds(w * KWIN, KWIN)], kbuf)

        def chunk(t, _):
            kv = kbuf[pl.ds(t * 16, 16)]
            iv = (w * KWIN + t * 16) + lanes
            _, last = plsc.scan_count(kv)
            m = last & (kv >= base) & (kv < hi)
            plsc.store_scatter(winner_v, [kv - base], iv, mask=m)
            return 0
        lax.fori_loop(0, KWIN // 16, chunk, 0)
        return 0
    lax.fori_loop(0, nkeys // KWIN, key_window, 0)

    def remap(t, off):
        wv = winner_v[pl.ds(t * 16, 16)]
        rowv = (base + t * 16) + lanes
        inv = wv < 0
        in_slab = rowv < hi
        ibuf[pl.ds(t * 16, 16)] = jnp.where(inv, rowv, wv)
        mm = inv & in_slab
        plsc.store_compressed(inv_flat.at[pl.ds(off, 16)], rowv, mask=mm)
        cnt = jnp.max(plsc.all_reduce_population_count(mm))
        return off + cnt
    cnt = lax.fori_loop(0, PER_W // 16, remap, jnp.int32(0))

    def gwin(w, _):
        st = base + w * WIN
        pltpu.async_copy(pe_hbm.at[ibuf.at[pl.ds(w * WIN, WIN)]],
                         zbuf, sem).wait()
        pltpu.sync_copy(zbuf, out_hbm.at[pl.ds(st, WIN), :])
        return 0
    lax.fori_loop(0, nrows // WIN, gwin, 0)

    def zrow(t, _):
        zbuf[t // 16, pl.ds((t % 16) * 16, 16)] = jnp.zeros((16,), jnp.float32)
        return 0
    lax.fori_loop(0, WIN * (D // 16), zrow, 0)

    zsrc = zbuf.at[pl.ds(0, 16), :]
    nfull = cnt // 16
    r = cnt % 16

    def zwin(w, _):
        v = inv_flat[pl.ds(w * 16, 16)]
        pltpu.async_copy(zsrc, out_hbm.at[v], sem).wait()
        return 0
    lax.fori_loop(0, nfull, zwin, 0)

    @pl.when(r > 0)
    def _tail():
        v = inv_flat[pl.ds(nfull * 16, 16)]
        bvec = lax.gather(
            v, jnp.full((16, 1), r - 1, jnp.int32),
            dimension_numbers=lax.GatherDimensionNumbers(
                offset_dims=(), collapsed_slice_dims=(0,),
                start_index_map=(0,)),
            slice_sizes=(1,),
            mode=lax.GatherScatterMode.PROMISE_IN_BOUNDS)
        vfinal = jnp.where(lanes < r, v, bvec)
        pltpu.async_copy(zsrc, out_hbm.at[vfinal], sem).wait()


@functools.partial(jax.jit, static_argnums=())
def _sc_call(keys, pe):
    rows = keys.shape[0]
    call = functools.partial(
        pl.kernel,
        out_type=jax.ShapeDtypeStruct((rows, D), jnp.float32),
        mesh=plsc.VectorSubcoreMesh(core_axis_name="c", subcore_axis_name="s"),
        compiler_params=pltpu.CompilerParams(needs_layout_passes=False),
        scratch_types=[
            pltpu.VMEM((PER_W,), jnp.int32),
            pltpu.VMEM((KWIN,), jnp.int32),
            pltpu.VMEM((PER_W,), jnp.int32),
            pltpu.VMEM((PER_W + WIN,), jnp.int32),
            pltpu.VMEM((WIN, D), jnp.float32),
            pltpu.SemaphoreType.DMA,
        ],
    )(_sc_kernel)
    return call(keys, pe)


def kernel(x, pe):
    rows = x.shape[0]
    n = rows // 2
    keys = x[:, 0] * 2 + x[:, 1]
    out = _sc_call(keys, pe)
    return out.reshape(n, 2 * D)
